# baseline (speedup 1.0000x reference)
import functools

import jax
import jax.numpy as jnp
from jax import lax
from jax.experimental import pallas as pl
from jax.experimental.pallas import tpu as pltpu
from jax.experimental.pallas import tpu_sc as plsc

B = 16384
F = 32
NQ = 4


RBLK = 8192


def _relayout_body(x_ref, o_ref):
  eye = jnp.eye(F, dtype=jnp.float32)
  x = x_ref[...]
  parts = []
  for a in range(NQ):
    parts.append(jax.lax.dot_general(
        x[:, a * RBLK:(a + 1) * RBLK], eye, (((0,), (0,)), ((), ())),
        preferred_element_type=jnp.float32))
  o_ref[...] = jnp.concatenate(parts, axis=1)


def _relayout(table):
  n = table.shape[0]
  tt = table.T
  nblk = pl.cdiv(n, NQ * RBLK)
  return pl.pallas_call(
      _relayout_body,
      grid=(nblk,),
      in_specs=[pl.BlockSpec((F, NQ * RBLK), lambda i: (0, i))],
      out_specs=pl.BlockSpec((RBLK, NQ * F), lambda i: (i, 0)),
      out_shape=jax.ShapeDtypeStruct((nblk * RBLK, NQ * F), jnp.float32),
      compiler_params=pltpu.CompilerParams(vmem_limit_bytes=100 * 1024 * 1024),
  )(tt)


CHUNK = 256


@functools.lru_cache(maxsize=None)
def _make_sc_gather(nc: int, ns: int, b_per_w: int):
  mesh = plsc.VectorSubcoreMesh(core_axis_name="c", subcore_axis_name="s")

  @functools.partial(
      pl.kernel,
      mesh=mesh,
      out_type=tuple(
          jax.ShapeDtypeStruct((B, NQ * F), jnp.float32) for _ in range(4)),
      scratch_types=[
          pltpu.VMEM((b_per_w,), jnp.int32),
          pltpu.VMEM((b_per_w,), jnp.int32),
          pltpu.VMEM((b_per_w,), jnp.int32),
          pltpu.VMEM((CHUNK, NQ * F), jnp.float32),
          pltpu.VMEM((CHUNK, NQ * F), jnp.float32),
          pltpu.SemaphoreType.DMA,
          pltpu.SemaphoreType.DMA,
      ],
      compiler_params=pltpu.CompilerParams(use_tc_tiling_on_sc=False),
  )
  def sc_gather(urow_hbm, mrow_hbm, irow_hbm, gu_hbm, gi_hbm, mu_hbm, mi_hbm,
                gu_out, gi_out, mu_out, mi_out,
                urow_v, mrow_v, irow_v, buf0_v, buf1_v, sem0, sem1):
    wid = lax.axis_index("s") * nc + lax.axis_index("c")
    base = wid * b_per_w
    pltpu.sync_copy(urow_hbm.at[pl.ds(base, b_per_w)], urow_v)
    pltpu.sync_copy(mrow_hbm.at[pl.ds(base, b_per_w)], mrow_v)
    pltpu.sync_copy(irow_hbm.at[pl.ds(base, b_per_w)], irow_v)

    n_chunks = b_per_w // CHUNK
    work = []
    for table, row_v, out in ((gu_hbm, urow_v, gu_out),
                              (gi_hbm, irow_v, gi_out),
                              (mu_hbm, mrow_v, mu_out),
                              (mi_hbm, irow_v, mi_out)):
      for c in range(n_chunks):
        work.append((table, row_v, out, c))

    bufs = (buf0_v, buf1_v)
    sems = (sem0, sem1)
    copies = [None, None]
    for k, (table, row_v, out, c) in enumerate(work):
      slot = k % 2
      if copies[slot] is not None:
        pt, pr, pout, pc, pcopy = copies[slot]
        pcopy.wait()
        pltpu.sync_copy(bufs[slot], pout.at[pl.ds(base + pc * CHUNK, CHUNK)])
      idx_slice = row_v.at[pl.ds(c * CHUNK, CHUNK)]
      cp = pltpu.async_copy(table.at[idx_slice], bufs[slot], sems[slot])
      copies[slot] = (table, row_v, out, c, cp)
    for slot in range(2):
      if copies[slot] is not None:
        pt, pr, pout, pc, pcopy = copies[slot]
        pcopy.wait()
        pltpu.sync_copy(bufs[slot], pout.at[pl.ds(base + pc * CHUNK, CHUNK)])

  return sc_gather


def _pick(w128, quarter):
  out = None
  for a in range(NQ):
    part = jnp.where(quarter == a, w128[:, a * F:(a + 1) * F], 0.0)
    out = part if out is None else out + part
  return out


def _tc_mlp_body(uq_ref, mq_ref, iq_ref, gu_ref, gi_ref, mu_ref, mi_ref,
                 w1a_ref, w1b_ref, b1_ref, w2_ref, b2_ref,
                 wog_ref, wom_ref, bo_ref, out_ref):
  uq = uq_ref[...]
  mq = mq_ref[...]
  iq = iq_ref[...]
  gu = _pick(gu_ref[...], uq)
  gi = _pick(gi_ref[...], iq)
  mu = _pick(mu_ref[...], mq)
  mi = _pick(mi_ref[...], iq)
  gmf = gu * gi
  h = jnp.dot(mu, w1a_ref[...], preferred_element_type=jnp.float32)
  h = h + jnp.dot(mi, w1b_ref[...], preferred_element_type=jnp.float32)
  h = jnp.maximum(h + b1_ref[...], 0.0)
  h2 = jnp.dot(h, w2_ref[...], preferred_element_type=jnp.float32)
  h2 = jnp.maximum(h2 + b2_ref[...], 0.0)
  out = jnp.sum(gmf * wog_ref[...], axis=1)
  out = out + jnp.sum(h2 * wom_ref[...], axis=1)
  out_ref[...] = out + bo_ref[0]


def _tc_mlp(uq, mq, iq, gu, gi, mu, mi, W1a, W1b, b1, W2, b2, wo_g, wo_m, bo):
  blk = 2048
  grid = (B // blk,)
  idx_spec = pl.BlockSpec((blk, 1), lambda i: (i, 0))
  row_spec = pl.BlockSpec((blk, NQ * F), lambda i: (i, 0))
  full = lambda shape: pl.BlockSpec(shape, lambda i: tuple(0 for _ in shape))
  return pl.pallas_call(
      _tc_mlp_body,
      grid=grid,
      in_specs=[
          idx_spec, idx_spec, idx_spec,
          row_spec, row_spec, row_spec, row_spec,
          full(W1a.shape), full(W1b.shape), full(b1.shape),
          full(W2.shape), full(b2.shape),
          full(wo_g.shape), full(wo_m.shape), full(bo.shape),
      ],
      out_specs=pl.BlockSpec((blk,), lambda i: (i,)),
      out_shape=jax.ShapeDtypeStruct((B,), jnp.float32),
  )(uq, mq, iq, gu, gi, mu, mi, W1a, W1b, b1, W2, b2, wo_g, wo_m, bo)


@jax.jit
def _neumf(user_idx, item_idx, gmf_user_emb, gmf_item_emb,
           mlp_user_emb, mlp_item_emb, W1, b1, W2, b2, Wo, bo):
  info = plsc.get_sparse_core_info()
  nw = info.num_cores * info.num_subcores
  sc = _make_sc_gather(info.num_cores, info.num_subcores, B // nw)

  uidx = user_idx.astype(jnp.int32)
  iidx = item_idx.astype(jnp.int32)
  uq = (uidx >> 13) & 3
  urow = ((uidx >> 15) << 13) + (uidx & 8191)
  iq = (iidx >> 13) & 3
  irow = ((iidx >> 15) << 13) + (iidx & 8191)

  mq = uidx & 3
  mrow = uidx >> 2
  mu_t = mlp_user_emb.reshape(-1, NQ * F)
  gu_t = _relayout(gmf_user_emb)
  gi_t = _relayout(gmf_item_emb)
  mi_t = _relayout(mlp_item_emb)

  gu, gi, mu, mi = sc(urow, mrow, irow, gu_t, gi_t, mu_t, mi_t)

  W1a, W1b = W1[:F], W1[F:]
  wo_g, wo_m = Wo[:F, 0], Wo[F:, 0]
  return _tc_mlp(uq.reshape(B, 1), mq.reshape(B, 1), iq.reshape(B, 1),
                 gu, gi, mu, mi, W1a, W1b, b1, W2, b2, wo_g, wo_m, bo)


def kernel(user_idx, item_idx, gmf_user_emb, gmf_item_emb,
           mlp_user_emb, mlp_item_emb, W1, b1, W2, b2, Wo, bo):
  return _neumf(user_idx, item_idx, gmf_user_emb, gmf_item_emb,
                mlp_user_emb, mlp_item_emb, W1, b1, W2, b2, Wo, bo)


# speedup vs baseline: 1.1405x; 1.1405x over previous
import functools

import jax
import jax.numpy as jnp
from jax import lax
from jax.experimental import pallas as pl
from jax.experimental.pallas import tpu as pltpu
from jax.experimental.pallas import tpu_sc as plsc

B = 16384
F = 32
NQ = 4


RBLK = 8192


def _relayout_body(x_ref, o_ref):
  eye = jnp.eye(F, dtype=jnp.float32)
  x = x_ref[...]
  parts = []
  for a in range(NQ):
    parts.append(jax.lax.dot_general(
        x[:, a * RBLK:(a + 1) * RBLK], eye, (((0,), (0,)), ((), ())),
        preferred_element_type=jnp.float32))
  o_ref[...] = jnp.concatenate(parts, axis=1)


def _relayout(table):
  n = table.shape[0]
  tt = table.T
  nblk = pl.cdiv(n, NQ * RBLK)
  return pl.pallas_call(
      _relayout_body,
      grid=(nblk,),
      in_specs=[pl.BlockSpec((F, NQ * RBLK), lambda i: (0, i))],
      out_specs=pl.BlockSpec((RBLK, NQ * F), lambda i: (i, 0)),
      out_shape=jax.ShapeDtypeStruct((nblk * RBLK, NQ * F), jnp.float32),
      compiler_params=pltpu.CompilerParams(vmem_limit_bytes=100 * 1024 * 1024),
  )(tt)


CHUNK = 256


@functools.lru_cache(maxsize=None)
def _make_sc_gather(nc: int, ns: int, b_per_w: int):
  mesh = plsc.VectorSubcoreMesh(core_axis_name="c", subcore_axis_name="s")

  @functools.partial(
      pl.kernel,
      mesh=mesh,
      out_type=tuple(
          jax.ShapeDtypeStruct((B, NQ * F), jnp.float32) for _ in range(4)),
      scratch_types=[
          pltpu.VMEM((b_per_w,), jnp.int32),
          pltpu.VMEM((b_per_w,), jnp.int32),
          pltpu.VMEM((b_per_w,), jnp.int32),
          pltpu.VMEM((CHUNK, NQ * F), jnp.float32),
          pltpu.VMEM((CHUNK, NQ * F), jnp.float32),
          pltpu.SemaphoreType.DMA,
          pltpu.SemaphoreType.DMA,
      ],
      compiler_params=pltpu.CompilerParams(use_tc_tiling_on_sc=False),
  )
  def sc_gather(urow_hbm, mrow_hbm, irow_hbm, gu_hbm, gi_hbm, mu_hbm, mi_hbm,
                gu_out, gi_out, mu_out, mi_out,
                urow_v, mrow_v, irow_v, buf0_v, buf1_v, sem0, sem1):
    wid = lax.axis_index("s") * nc + lax.axis_index("c")
    base = wid * b_per_w
    pltpu.sync_copy(urow_hbm.at[pl.ds(base, b_per_w)], urow_v)
    pltpu.sync_copy(mrow_hbm.at[pl.ds(base, b_per_w)], mrow_v)
    pltpu.sync_copy(irow_hbm.at[pl.ds(base, b_per_w)], irow_v)

    n_chunks = b_per_w // CHUNK
    work = []
    for table, row_v, out in ((gu_hbm, urow_v, gu_out),
                              (gi_hbm, irow_v, gi_out),
                              (mu_hbm, mrow_v, mu_out),
                              (mi_hbm, irow_v, mi_out)):
      for c in range(n_chunks):
        work.append((table, row_v, out, c))

    bufs = (buf0_v, buf1_v)
    sems = (sem0, sem1)
    copies = [None, None]
    for k, (table, row_v, out, c) in enumerate(work):
      slot = k % 2
      if copies[slot] is not None:
        pt, pr, pout, pc, pcopy = copies[slot]
        pcopy.wait()
        pltpu.sync_copy(bufs[slot], pout.at[pl.ds(base + pc * CHUNK, CHUNK)])
      idx_slice = row_v.at[pl.ds(c * CHUNK, CHUNK)]
      cp = pltpu.async_copy(table.at[idx_slice], bufs[slot], sems[slot])
      copies[slot] = (table, row_v, out, c, cp)
    for slot in range(2):
      if copies[slot] is not None:
        pt, pr, pout, pc, pcopy = copies[slot]
        pcopy.wait()
        pltpu.sync_copy(bufs[slot], pout.at[pl.ds(base + pc * CHUNK, CHUNK)])

  return sc_gather


def _pick(w128, quarter):
  out = None
  for a in range(NQ):
    part = jnp.where(quarter == a, w128[:, a * F:(a + 1) * F], 0.0)
    out = part if out is None else out + part
  return out


def _tc_mlp_body(uq_ref, mq_ref, iq_ref, gu_ref, gi_ref, mu_ref, mi_ref,
                 w1a_ref, w1b_ref, b1_ref, w2_ref, b2_ref,
                 wog_ref, wom_ref, bo_ref, out_ref):
  uq = uq_ref[...]
  mq = mq_ref[...]
  iq = iq_ref[...]
  gu = _pick(gu_ref[...], uq)
  gi = _pick(gi_ref[...], iq)
  mu = _pick(mu_ref[...], mq)
  mi = _pick(mi_ref[...], iq)
  gmf = gu * gi
  h = jnp.dot(mu, w1a_ref[...], preferred_element_type=jnp.float32)
  h = h + jnp.dot(mi, w1b_ref[...], preferred_element_type=jnp.float32)
  h = jnp.maximum(h + b1_ref[...], 0.0)
  h2 = jnp.dot(h, w2_ref[...], preferred_element_type=jnp.float32)
  h2 = jnp.maximum(h2 + b2_ref[...], 0.0)
  out = jnp.sum(gmf * wog_ref[...], axis=1)
  out = out + jnp.sum(h2 * wom_ref[...], axis=1)
  out_ref[...] = out + bo_ref[0]


def _tc_mlp(uq, mq, iq, gu, gi, mu, mi, W1a, W1b, b1, W2, b2, wo_g, wo_m, bo):
  blk = 2048
  grid = (B // blk,)
  idx_spec = pl.BlockSpec((blk, 1), lambda i: (i, 0))
  row_spec = pl.BlockSpec((blk, NQ * F), lambda i: (i, 0))
  full = lambda shape: pl.BlockSpec(shape, lambda i: tuple(0 for _ in shape))
  return pl.pallas_call(
      _tc_mlp_body,
      grid=grid,
      in_specs=[
          idx_spec, idx_spec, idx_spec,
          row_spec, row_spec, row_spec, row_spec,
          full(W1a.shape), full(W1b.shape), full(b1.shape),
          full(W2.shape), full(b2.shape),
          full(wo_g.shape), full(wo_m.shape), full(bo.shape),
      ],
      out_specs=pl.BlockSpec((blk,), lambda i: (i,)),
      out_shape=jax.ShapeDtypeStruct((B,), jnp.float32),
  )(uq, mq, iq, gu, gi, mu, mi, W1a, W1b, b1, W2, b2, wo_g, wo_m, bo)


@jax.jit
def _neumf(user_idx, item_idx, gmf_user_emb, gmf_item_emb,
           mlp_user_emb, mlp_item_emb, W1, b1, W2, b2, Wo, bo):
  info = plsc.get_sparse_core_info()
  nw = info.num_cores * info.num_subcores
  sc = _make_sc_gather(info.num_cores, info.num_subcores, B // nw)

  uidx = user_idx.astype(jnp.int32)
  iidx = item_idx.astype(jnp.int32)
  uq = (uidx >> 13) & 3
  urow = ((uidx >> 15) << 13) + (uidx & 8191)
  iq = (iidx >> 13) & 3
  irow = ((iidx >> 15) << 13) + (iidx & 8191)

  gu_t = _relayout(gmf_user_emb)
  gi_t = _relayout(gmf_item_emb)
  mu_t = _relayout(mlp_user_emb)
  mi_t = _relayout(mlp_item_emb)

  gu, gi, mu, mi = sc(urow, urow, irow, gu_t, gi_t, mu_t, mi_t)

  W1a, W1b = W1[:F], W1[F:]
  wo_g, wo_m = Wo[:F, 0], Wo[F:, 0]
  return _tc_mlp(uq.reshape(B, 1), uq.reshape(B, 1), iq.reshape(B, 1),
                 gu, gi, mu, mi, W1a, W1b, b1, W2, b2, wo_g, wo_m, bo)


def kernel(user_idx, item_idx, gmf_user_emb, gmf_item_emb,
           mlp_user_emb, mlp_item_emb, W1, b1, W2, b2, Wo, bo):
  return _neumf(user_idx, item_idx, gmf_user_emb, gmf_item_emb,
                mlp_user_emb, mlp_item_emb, W1, b1, W2, b2, Wo, bo)
